# baseline (speedup 1.0000x reference)
import functools

import jax
import jax.numpy as jnp
from jax import lax
from jax.experimental import pallas as pl
from jax.experimental.pallas import tpu as pltpu
from jax.experimental.pallas import tpu_sc as plsc

N = 100000
N_PAD = 100352
E = 6400000
NC, NS = 2, 16
BATCH = 512
EPT = 202752
E_PAD = EPT * NC * NS
IDX_ROWS_PT = EPT // BATCH
M_AGG = 3
M_DEG = 6
NBODY_AGG = EPT // (BATCH * M_AGG)
NBODY_DEG = EPT // (BATCH * M_DEG)
ROWS_PT = N_PAD // NS

_mesh = plsc.VectorSubcoreMesh(core_axis_name="c", subcore_axis_name="s")
_sc_params = pltpu.CompilerParams(use_tc_tiling_on_sc=False)


def _deg_body(dst_hbm, z_hbm, out_hbm, acc, idx, ones, sem_i, sem_s):
    c = lax.axis_index("c")
    s = lax.axis_index("s")
    rowbase = (c * NS + s) * IDX_ROWS_PT
    r0 = s * ROWS_PT
    for i in range(BATCH // 16):
        ones[pl.ds(i * 16, 16)] = jnp.full((16,), 1.0, jnp.float32)
    pltpu.sync_copy(z_hbm, acc.at[pl.ds(r0, ROWS_PT)])
    plsc.subcore_barrier()

    def _drain_scatters():
        for j in range(M_DEG):
            pltpu.make_async_copy(ones, acc.at[idx.at[j]], sem_s).wait()

    @pl.loop(0, NBODY_DEG)
    def _(i):
        base = rowbase + i * M_DEG

        @pl.when(i > 0)
        def _drain():
            _drain_scatters()

        pltpu.async_copy(dst_hbm.at[pl.ds(base, M_DEG)], idx, sem_i).wait()
        for j in range(M_DEG):
            pltpu.async_copy(ones, acc.at[idx.at[j]], sem_s, add=True)

    _drain_scatters()
    plsc.subcore_barrier()
    pltpu.sync_copy(acc.at[pl.ds(r0, ROWS_PT)], out_hbm.at[c, pl.ds(r0, ROWS_PT)])


_deg_kernel = pl.kernel(
    _deg_body,
    out_type=jax.ShapeDtypeStruct((NC, N_PAD), jnp.float32),
    mesh=_mesh,
    compiler_params=_sc_params,
    scratch_types=[
        pltpu.VMEM_SHARED((N_PAD,), jnp.float32),
        pltpu.VMEM((M_DEG, BATCH), jnp.int32),
        pltpu.VMEM((BATCH,), jnp.float32),
        pltpu.SemaphoreType.DMA,
        pltpu.SemaphoreType.DMA,
    ],
)


def _agg_body(F, h_hbm, src_hbm, dst_hbm, z_hbm, out_hbm,
              acc, sbuf, dbuf, rows, sem_i, sem_g, sem_s):
    c = lax.axis_index("c")
    s = lax.axis_index("s")
    rowbase = (c * NS + s) * IDX_ROWS_PT
    r0 = s * ROWS_PT
    pltpu.sync_copy(z_hbm, acc.at[pl.ds(r0, ROWS_PT)])
    plsc.subcore_barrier()

    def _drain_scatters():
        for j in range(M_AGG):
            pltpu.make_async_copy(rows.at[j], acc.at[dbuf.at[j]], sem_s).wait()

    @pl.loop(0, NBODY_AGG)
    def _(i):
        base = rowbase + i * M_AGG

        @pl.when(i > 0)
        def _drain():
            _drain_scatters()

        cs = pltpu.async_copy(src_hbm.at[pl.ds(base, M_AGG)], sbuf, sem_i)
        cd = pltpu.async_copy(dst_hbm.at[pl.ds(base, M_AGG)], dbuf, sem_i)
        cs.wait()
        cd.wait()
        gs = [
            pltpu.async_copy(h_hbm.at[sbuf.at[j]], rows.at[j], sem_g)
            for j in range(M_AGG)
        ]
        for j in range(M_AGG):
            gs[j].wait()
            pltpu.async_copy(rows.at[j], acc.at[dbuf.at[j]], sem_s, add=True)

    _drain_scatters()
    plsc.subcore_barrier()
    pltpu.sync_copy(acc.at[pl.ds(r0, ROWS_PT)],
                    out_hbm.at[c, pl.ds(r0, ROWS_PT)])


def _make_agg(F):
    return pl.kernel(
        functools.partial(_agg_body, F),
        out_type=jax.ShapeDtypeStruct((NC, N_PAD, F), jnp.float32),
        mesh=_mesh,
        compiler_params=_sc_params,
        scratch_types=[
            pltpu.VMEM_SHARED((N_PAD, F), jnp.float32),
            pltpu.VMEM((M_AGG, BATCH), jnp.int32),
            pltpu.VMEM((M_AGG, BATCH), jnp.int32),
            pltpu.VMEM((M_AGG, BATCH, F), jnp.float32),
            pltpu.SemaphoreType.DMA,
            pltpu.SemaphoreType.DMA,
            pltpu.SemaphoreType.DMA,
        ],
    )


_agg16 = _make_agg(16)
_agg8 = _make_agg(8)


def _tc2_body(degp_ref, x_ref, w1_ref, h_ref, d_ref):
    deg = degp_ref[0, :] + degp_ref[1, :] + 1.0
    dis = lax.rsqrt(deg)[:, None]
    h_ref[...] = jnp.dot(x_ref[...], w1_ref[...],
                         preferred_element_type=jnp.float32) * dis
    d_ref[...] = dis


def _tc4_body(aggp_ref, h1_ref, d_ref, b1_ref, w2_ref, h2_ref):
    d = d_ref[...]
    ssum = aggp_ref[0] + aggp_ref[1] + h1_ref[...]
    h = jnp.maximum(ssum * d + b1_ref[...], 0.0)
    h2 = jnp.dot(h, w2_ref[...], preferred_element_type=jnp.float32) * d
    h2_ref[...] = jnp.pad(h2, ((0, 0), (0, 6)))


def _tc6_body(aggp_ref, h2_ref, d_ref, b2_ref, o_ref):
    ssum = aggp_ref[0] + aggp_ref[1] + h2_ref[...]
    o_ref[...] = ssum[:, :2] * d_ref[...] + b2_ref[...]


_B2 = 1024
_tc2 = pl.pallas_call(
    _tc2_body,
    out_shape=(jax.ShapeDtypeStruct((N_PAD, 16), jnp.float32),
               jax.ShapeDtypeStruct((N_PAD, 1), jnp.float32)),
    grid=(N_PAD // _B2,),
    in_specs=[
        pl.BlockSpec((NC, _B2), lambda i: (0, i)),
        pl.BlockSpec((_B2, 10), lambda i: (i, 0)),
        pl.BlockSpec((10, 16), lambda i: (0, 0)),
    ],
    out_specs=(pl.BlockSpec((_B2, 16), lambda i: (i, 0)),
               pl.BlockSpec((_B2, 1), lambda i: (i, 0))),
)

_tc4 = pl.pallas_call(
    _tc4_body,
    out_shape=jax.ShapeDtypeStruct((N_PAD, 8), jnp.float32),
    grid=(N_PAD // _B2,),
    in_specs=[
        pl.BlockSpec((NC, _B2, 16), lambda i: (0, i, 0)),
        pl.BlockSpec((_B2, 16), lambda i: (i, 0)),
        pl.BlockSpec((_B2, 1), lambda i: (i, 0)),
        pl.BlockSpec((1, 16), lambda i: (0, 0)),
        pl.BlockSpec((16, 2), lambda i: (0, 0)),
    ],
    out_specs=pl.BlockSpec((_B2, 8), lambda i: (i, 0)),
)

_B6 = 2000
_tc6 = pl.pallas_call(
    _tc6_body,
    out_shape=jax.ShapeDtypeStruct((N, 2), jnp.float32),
    grid=(N // _B6,),
    in_specs=[
        pl.BlockSpec((NC, _B6, 8), lambda i: (0, i, 0)),
        pl.BlockSpec((_B6, 8), lambda i: (i, 0)),
        pl.BlockSpec((_B6, 1), lambda i: (i, 0)),
        pl.BlockSpec((1, 2), lambda i: (0, 0)),
    ],
    out_specs=pl.BlockSpec((_B6, 2), lambda i: (i, 0)),
)


def kernel(x, edge_index, W1, b1, W2, b2):
    src = edge_index[0].astype(jnp.int32)
    dst = edge_index[1].astype(jnp.int32)
    fill = jnp.full((E_PAD - E,), N, jnp.int32)
    src2d = jnp.concatenate([src, fill]).reshape(E_PAD // BATCH, BATCH)
    dst2d = jnp.concatenate([dst, fill]).reshape(E_PAD // BATCH, BATCH)
    x_pad = jnp.pad(x, ((0, N_PAD - N), (0, 0)))

    z1 = jnp.zeros((ROWS_PT,), jnp.float32)
    z16 = jnp.zeros((ROWS_PT, 16), jnp.float32)
    z8 = jnp.zeros((ROWS_PT, 8), jnp.float32)

    degp = _deg_kernel(dst2d, z1)
    h1p, d = _tc2(degp, x_pad, W1)
    aggp1 = _agg16(h1p, src2d, dst2d, z16)
    h2p = _tc4(aggp1, h1p, d, b1.reshape(1, 16), W2)
    aggp2 = _agg8(h2p, src2d, dst2d, z8)
    out = _tc6(aggp2, h2p, d, b2.reshape(1, 2))
    return out


# speedup vs baseline: 1.2497x; 1.2497x over previous
import functools

import jax
import jax.numpy as jnp
from jax import lax
from jax.experimental import pallas as pl
from jax.experimental.pallas import tpu as pltpu
from jax.experimental.pallas import tpu_sc as plsc

N = 100000
N_PAD = 100352
E = 6400000
NC, NS = 2, 16
BATCH = 128
EPT = 202752
E_PAD = EPT * NC * NS
IDX_ROWS_PT = EPT // BATCH
M_AGG = 12
M_DEG = 16
NBODY_AGG = EPT // (BATCH * M_AGG)
NBODY_DEG = EPT // (BATCH * M_DEG)
ROWS_PT = N_PAD // NS

_mesh = plsc.VectorSubcoreMesh(core_axis_name="c", subcore_axis_name="s")
_sc_params = pltpu.CompilerParams(use_tc_tiling_on_sc=False)


def _deg_body(dst_hbm, z_hbm, out_hbm, acc, idx, ones, sem_i, sem_s):
    c = lax.axis_index("c")
    s = lax.axis_index("s")
    rowbase = (c * NS + s) * IDX_ROWS_PT
    r0 = s * ROWS_PT
    for i in range(BATCH // 16):
        ones[pl.ds(i * 16, 16)] = jnp.full((16,), 1.0, jnp.float32)
    pltpu.sync_copy(z_hbm, acc.at[pl.ds(r0, ROWS_PT)])
    plsc.subcore_barrier()

    def _drain_scatters():
        for j in range(M_DEG):
            pltpu.make_async_copy(ones, acc.at[idx.at[j]], sem_s).wait()

    @pl.loop(0, NBODY_DEG)
    def _(i):
        base = rowbase + i * M_DEG

        @pl.when(i > 0)
        def _drain():
            _drain_scatters()

        pltpu.async_copy(dst_hbm.at[pl.ds(base, M_DEG)], idx, sem_i).wait()
        for j in range(M_DEG):
            pltpu.async_copy(ones, acc.at[idx.at[j]], sem_s, add=True)

    _drain_scatters()
    plsc.subcore_barrier()
    pltpu.sync_copy(acc.at[pl.ds(r0, ROWS_PT)], out_hbm.at[c, pl.ds(r0, ROWS_PT)])


_deg_kernel = pl.kernel(
    _deg_body,
    out_type=jax.ShapeDtypeStruct((NC, N_PAD), jnp.float32),
    mesh=_mesh,
    compiler_params=_sc_params,
    scratch_types=[
        pltpu.VMEM_SHARED((N_PAD,), jnp.float32),
        pltpu.VMEM((M_DEG, BATCH), jnp.int32),
        pltpu.VMEM((BATCH,), jnp.float32),
        pltpu.SemaphoreType.DMA,
        pltpu.SemaphoreType.DMA,
    ],
)


def _agg_body(F, spmem_table, h_hbm, src_hbm, dst_hbm, z_hbm, out_hbm,
              acc, tab, sbuf, dbuf, rows, sem_i, sem_g, sem_s):
    c = lax.axis_index("c")
    s = lax.axis_index("s")
    rowbase = (c * NS + s) * IDX_ROWS_PT
    r0 = s * ROWS_PT
    pltpu.sync_copy(z_hbm, acc.at[pl.ds(r0, ROWS_PT)])
    if spmem_table:
        pltpu.sync_copy(h_hbm.at[pl.ds(r0, ROWS_PT)], tab.at[pl.ds(r0, ROWS_PT)])
        src_tab = tab
    else:
        src_tab = h_hbm
    plsc.subcore_barrier()

    def _drain_scatters():
        for j in range(M_AGG):
            pltpu.make_async_copy(rows.at[j], acc.at[dbuf.at[j]], sem_s).wait()

    @pl.loop(0, NBODY_AGG)
    def _(i):
        base = rowbase + i * M_AGG

        @pl.when(i > 0)
        def _drain():
            _drain_scatters()

        cs = pltpu.async_copy(src_hbm.at[pl.ds(base, M_AGG)], sbuf, sem_i)
        cd = pltpu.async_copy(dst_hbm.at[pl.ds(base, M_AGG)], dbuf, sem_i)
        cs.wait()
        cd.wait()
        gs = [
            pltpu.async_copy(src_tab.at[sbuf.at[j]], rows.at[j], sem_g)
            for j in range(M_AGG)
        ]
        for j in range(M_AGG):
            gs[j].wait()
            pltpu.async_copy(rows.at[j], acc.at[dbuf.at[j]], sem_s, add=True)

    _drain_scatters()
    plsc.subcore_barrier()
    pltpu.sync_copy(acc.at[pl.ds(r0, ROWS_PT)],
                    out_hbm.at[c, pl.ds(r0, ROWS_PT)])


def _make_agg(F, spmem_table):
    tab_shape = (N_PAD, F) if spmem_table else (8,)
    return pl.kernel(
        functools.partial(_agg_body, F, spmem_table),
        out_type=jax.ShapeDtypeStruct((NC, N_PAD, F), jnp.float32),
        mesh=_mesh,
        compiler_params=_sc_params,
        scratch_types=[
            pltpu.VMEM_SHARED((N_PAD, F), jnp.float32),
            pltpu.VMEM_SHARED(tab_shape, jnp.float32),
            pltpu.VMEM((M_AGG, BATCH), jnp.int32),
            pltpu.VMEM((M_AGG, BATCH), jnp.int32),
            pltpu.VMEM((M_AGG, BATCH, F), jnp.float32),
            pltpu.SemaphoreType.DMA,
            pltpu.SemaphoreType.DMA,
            pltpu.SemaphoreType.DMA,
        ],
    )


_agg16 = _make_agg(16, spmem_table=False)
_agg8 = _make_agg(8, spmem_table=True)


def _tc2_body(degp_ref, x_ref, w1_ref, h_ref, d_ref):
    deg = degp_ref[0, :] + degp_ref[1, :] + 1.0
    dis = lax.rsqrt(deg)[:, None]
    h_ref[...] = jnp.dot(x_ref[...], w1_ref[...],
                         preferred_element_type=jnp.float32) * dis
    d_ref[...] = dis


def _tc4_body(aggp_ref, h1_ref, d_ref, b1_ref, w2_ref, h2_ref):
    d = d_ref[...]
    ssum = aggp_ref[0] + aggp_ref[1] + h1_ref[...]
    h = jnp.maximum(ssum * d + b1_ref[...], 0.0)
    h2 = jnp.dot(h, w2_ref[...], preferred_element_type=jnp.float32) * d
    h2_ref[...] = jnp.pad(h2, ((0, 0), (0, 6)))


def _tc6_body(aggp_ref, h2_ref, d_ref, b2_ref, o_ref):
    ssum = aggp_ref[0] + aggp_ref[1] + h2_ref[...]
    o_ref[...] = ssum[:, :2] * d_ref[...] + b2_ref[...]


_B2 = 1024
_tc2 = pl.pallas_call(
    _tc2_body,
    out_shape=(jax.ShapeDtypeStruct((N_PAD, 16), jnp.float32),
               jax.ShapeDtypeStruct((N_PAD, 1), jnp.float32)),
    grid=(N_PAD // _B2,),
    in_specs=[
        pl.BlockSpec((NC, _B2), lambda i: (0, i)),
        pl.BlockSpec((_B2, 10), lambda i: (i, 0)),
        pl.BlockSpec((10, 16), lambda i: (0, 0)),
    ],
    out_specs=(pl.BlockSpec((_B2, 16), lambda i: (i, 0)),
               pl.BlockSpec((_B2, 1), lambda i: (i, 0))),
)

_tc4 = pl.pallas_call(
    _tc4_body,
    out_shape=jax.ShapeDtypeStruct((N_PAD, 8), jnp.float32),
    grid=(N_PAD // _B2,),
    in_specs=[
        pl.BlockSpec((NC, _B2, 16), lambda i: (0, i, 0)),
        pl.BlockSpec((_B2, 16), lambda i: (i, 0)),
        pl.BlockSpec((_B2, 1), lambda i: (i, 0)),
        pl.BlockSpec((1, 16), lambda i: (0, 0)),
        pl.BlockSpec((16, 2), lambda i: (0, 0)),
    ],
    out_specs=pl.BlockSpec((_B2, 8), lambda i: (i, 0)),
)

_B6 = 2000
_tc6 = pl.pallas_call(
    _tc6_body,
    out_shape=jax.ShapeDtypeStruct((N, 2), jnp.float32),
    grid=(N // _B6,),
    in_specs=[
        pl.BlockSpec((NC, _B6, 8), lambda i: (0, i, 0)),
        pl.BlockSpec((_B6, 8), lambda i: (i, 0)),
        pl.BlockSpec((_B6, 1), lambda i: (i, 0)),
        pl.BlockSpec((1, 2), lambda i: (0, 0)),
    ],
    out_specs=pl.BlockSpec((_B6, 2), lambda i: (i, 0)),
)


def kernel(x, edge_index, W1, b1, W2, b2):
    src = edge_index[0].astype(jnp.int32)
    dst = edge_index[1].astype(jnp.int32)
    fill = jnp.full((E_PAD - E,), N, jnp.int32)
    src2d = jnp.concatenate([src, fill]).reshape(E_PAD // BATCH, BATCH)
    dst2d = jnp.concatenate([dst, fill]).reshape(E_PAD // BATCH, BATCH)
    x_pad = jnp.pad(x, ((0, N_PAD - N), (0, 0)))

    z1 = jnp.zeros((ROWS_PT,), jnp.float32)
    z16 = jnp.zeros((ROWS_PT, 16), jnp.float32)
    z8 = jnp.zeros((ROWS_PT, 8), jnp.float32)

    degp = _deg_kernel(dst2d, z1)
    h1p, d = _tc2(degp, x_pad, W1)
    aggp1 = _agg16(h1p, src2d, dst2d, z16)
    h2p = _tc4(aggp1, h1p, d, b1.reshape(1, 16), W2)
    aggp2 = _agg8(h2p, src2d, dst2d, z8)
    out = _tc6(aggp2, h2p, d, b2.reshape(1, 2))
    return out
